# baseline (speedup 1.0000x reference)
import jax
import jax.numpy as jnp
from jax.experimental import pallas as pl
from jax.experimental.pallas import tpu as pltpu

_B, _H, _S, _D = 16, 16, 4096, 64
_L = 16
_BH = _B * _H
_SB = _S // 2
_G = 4


def _body(pos_ref, kc, vc, kv, vv, ko, vo):
    ko[...] = kc[...]
    vo[...] = vc[...]
    for sub in range(_G):
        for l in range(_L):
            p = pos_ref[l]
            r = p // 2
            lsrc = (slice(l // 2, l // 2 + 1), slice((l % 2) * _D, (l % 2) * _D + _D))
            for parity in (0, 1):
                dst = (pl.ds(r, 1), slice(parity * _D, parity * _D + _D))

                @pl.when(p % 2 == parity)
                def _():
                    ko[sub, dst[0], dst[1]] = kv[sub, lsrc[0], lsrc[1]]
                    vo[sub, dst[0], dst[1]] = vv[sub, lsrc[0], lsrc[1]]


def kernel(k_cache, v_cache, input_pos, k_val, v_val):
    kc = k_cache.reshape(_BH, _SB, 128)
    vc = v_cache.reshape(_BH, _SB, 128)
    kvl = k_val.reshape(_BH, _L // 2, 128)
    vvl = v_val.reshape(_BH, _L // 2, 128)
    grid = (_BH // _G,)
    cache_spec = pl.BlockSpec((_G, _SB, 128), lambda i, pos: (i, 0, 0))
    val_spec = pl.BlockSpec((_G, _L // 2, 128), lambda i, pos: (i, 0, 0))
    ko, vo = pl.pallas_call(
        _body,
        grid_spec=pltpu.PrefetchScalarGridSpec(
            num_scalar_prefetch=1,
            grid=grid,
            in_specs=[cache_spec, cache_spec, val_spec, val_spec],
            out_specs=[cache_spec, cache_spec],
        ),
        out_shape=[jax.ShapeDtypeStruct((_BH, _SB, 128), jnp.float32)] * 2,
        compiler_params=pltpu.CompilerParams(
            dimension_semantics=("parallel",),
        ),
    )(input_pos, kc, vc, kvl, vvl)
    return ko.reshape(_B, _H, _S, _D), vo.reshape(_B, _H, _S, _D)


# speedup vs baseline: 5.5698x; 5.5698x over previous
import jax
import jax.numpy as jnp
from jax.experimental import pallas as pl
from jax.experimental.pallas import tpu as pltpu

_B, _H, _S, _D = 16, 16, 4096, 64
_L = 16
_BH = _B * _H
_G = 2


def _body(kc, vc, kv, vv, oh, cm, ko, vo):
    mask = cm[...] > 0
    for g in range(_G):
        dk = jax.lax.dot(
            kv[g], oh[...], precision=jax.lax.Precision.HIGHEST,
            preferred_element_type=jnp.float32,
        )
        ko[g] = jnp.where(mask, dk, kc[g])
        dv = jax.lax.dot(
            vv[g], oh[...], precision=jax.lax.Precision.HIGHEST,
            preferred_element_type=jnp.float32,
        )
        vo[g] = jnp.where(mask, dv, vc[g])


def kernel(k_cache, v_cache, input_pos, k_val, v_val):
    kct = jnp.swapaxes(k_cache, 2, 3).reshape(_BH, _D, _S)
    vct = jnp.swapaxes(v_cache, 2, 3).reshape(_BH, _D, _S)
    kvt = jnp.swapaxes(k_val, 2, 3).reshape(_BH, _D, _L)
    vvt = jnp.swapaxes(v_val, 2, 3).reshape(_BH, _D, _L)

    nxt = jnp.concatenate([input_pos[1:], jnp.full((1,), -1, jnp.int32)])
    alive = input_pos != nxt
    cols = jax.lax.iota(jnp.int32, _S)
    onehot = (
        (input_pos[:, None] == cols[None, :]) & alive[:, None]
    ).astype(jnp.float32)
    colmask = jnp.sum(onehot, axis=0, keepdims=True)

    grid = (_BH // _G,)
    cache_spec = pl.BlockSpec((_G, _D, _S), lambda i: (i, 0, 0))
    val_spec = pl.BlockSpec((_G, _D, _L), lambda i: (i, 0, 0))
    oh_spec = pl.BlockSpec((_L, _S), lambda i: (0, 0))
    cm_spec = pl.BlockSpec((1, _S), lambda i: (0, 0))
    ko, vo = pl.pallas_call(
        _body,
        grid=grid,
        in_specs=[cache_spec, cache_spec, val_spec, val_spec, oh_spec, cm_spec],
        out_specs=[cache_spec, cache_spec],
        out_shape=[jax.ShapeDtypeStruct((_BH, _D, _S), jnp.float32)] * 2,
        compiler_params=pltpu.CompilerParams(
            dimension_semantics=("parallel",),
        ),
    )(kct, vct, kvt, vvt, onehot, colmask)
    ko = jnp.swapaxes(ko.reshape(_B, _H, _D, _S), 2, 3)
    vo = jnp.swapaxes(vo.reshape(_B, _H, _D, _S), 2, 3)
    return ko, vo
